# Initial kernel scaffold; baseline (speedup 1.0000x reference)
import jax
import jax.numpy as jnp
from jax.experimental import pallas as pl


def kernel(pred_tensor, target_boxes, target_cls, has_object_map):
    raise NotImplementedError("write your pallas kernel here")


# speedup vs baseline: 1.2537x; 1.2537x over previous
import jax
import jax.numpy as jnp
from jax.experimental import pallas as pl

S = 14
B = 2
L_COORD = 5.0
L_NOOBJ = 0.5
N_CLS = 20

_ROWS = 392
_COLS = 128


def _loss_kernel(pred_ref, tbox_ref, tcls_ref, mask_ref, out_ref):
    mask = mask_ref[0]
    no_mask = 1.0 - mask
    n_cells = float(_ROWS * _COLS)
    n_obj = jnp.maximum(jnp.sum(mask), 1.0)
    n_noobj = jnp.maximum(n_cells - jnp.sum(mask), 1.0)

    logit0 = pred_ref[B * 5]
    m = logit0
    for c in range(1, N_CLS):
        m = jnp.maximum(m, pred_ref[B * 5 + c])
    sumexp = jnp.exp(logit0 - m)
    best_t = tcls_ref[0]
    sel = logit0
    for c in range(1, N_CLS):
        x = pred_ref[B * 5 + c]
        sumexp = sumexp + jnp.exp(x - m)
        t = tcls_ref[c]
        upd = t > best_t
        best_t = jnp.where(upd, t, best_t)
        sel = jnp.where(upd, x, sel)
    ce = m + jnp.log(sumexp) - sel
    cls_loss = jnp.sum(mask * ce) / n_obj

    conf0 = pred_ref[4]
    conf1 = pred_ref[9]
    no_obj_loss = (jnp.sum(no_mask * conf0 * conf0)
                   + jnp.sum(no_mask * conf1 * conf1)) / n_noobj

    inv_s = 1.0 / S
    tx, ty, tw, th = tbox_ref[0], tbox_ref[1], tbox_ref[2], tbox_ref[3]
    tx1 = tx * inv_s - 0.5 * tw
    ty1 = ty * inv_s - 0.5 * th
    tx2 = tx * inv_s + 0.5 * tw
    ty2 = ty * inv_s + 0.5 * th
    t_area = (tx2 - tx1) * (ty2 - ty1)

    def box(b):
        x, y, w, h = pred_ref[5 * b], pred_ref[5 * b + 1], pred_ref[5 * b + 2], pred_ref[5 * b + 3]
        x1 = x * inv_s - 0.5 * w
        y1 = y * inv_s - 0.5 * h
        x2 = x * inv_s + 0.5 * w
        y2 = y * inv_s + 0.5 * h
        ix = jnp.clip(jnp.minimum(x2, tx2) - jnp.maximum(x1, tx1), 0.0, None)
        iy = jnp.clip(jnp.minimum(y2, ty2) - jnp.maximum(y1, ty1), 0.0, None)
        inter = ix * iy
        union = (x2 - x1) * (y2 - y1) + t_area - inter
        iou = inter / jnp.maximum(union, 1e-9)
        return (x1, y1, x2, y2), iou

    (b0, iou0), (b1, iou1) = box(0), box(1)
    upd = iou1 > iou0
    best_iou = jnp.where(upd, iou1, iou0)
    best_conf = jnp.where(upd, conf1, conf0)

    reg_loss = jnp.zeros_like(mask)
    for p0, p1, t in zip(b0, b1, (tx1, ty1, tx2, ty2)):
        d = jnp.where(upd, p1, p0) - t
        reg_loss = reg_loss + d * d
    reg_loss = jnp.sum(mask * reg_loss)

    dcf = best_conf - best_iou
    contain_loss = jnp.sum(mask * dcf * dcf)

    inv_n = 1.0 / 256.0
    total = inv_n * (L_COORD * reg_loss + contain_loss
                     + L_NOOBJ * no_obj_loss + cls_loss)
    out_ref[:, :] = jnp.broadcast_to(total, (1, 1))


def kernel(pred_tensor, target_boxes, target_cls, has_object_map):
    n = pred_tensor.shape[0]
    m = n * S * S
    pred_t = pred_tensor.reshape(m, B * 5 + N_CLS).T.reshape(B * 5 + N_CLS, _ROWS, _COLS)
    tbox_t = target_boxes.reshape(m, 4).T.reshape(4, _ROWS, _COLS)
    tcls_t = target_cls.reshape(m, N_CLS).T.reshape(N_CLS, _ROWS, _COLS)
    mask_t = has_object_map.astype(jnp.float32).reshape(1, _ROWS, _COLS)

    out = pl.pallas_call(
        _loss_kernel,
        out_shape=jax.ShapeDtypeStruct((1, 1), jnp.float32),
    )(pred_t, tbox_t, tcls_t, mask_t)
    return out[0, 0]
